# baseline (speedup 1.0000x reference)
import functools

import jax
import jax.numpy as jnp
from jax import lax
from jax.experimental import pallas as pl
from jax.experimental.pallas import tpu as pltpu
from jax.experimental.pallas import tpu_sc as plsc

D = 64
B = 16384 * 50
NC = 2
NS = 16
NW = NC * NS
BPW = B // NW
CH = 128
NCH = BPW // CH
PV = 500000
PCH = CH // 2

_mesh = plsc.VectorSubcoreMesh(core_axis_name="c", subcore_axis_name="s")


def _gather_body(idx_hbm, table_hbm, out_hbm,
                 idx_v, hidx0, hidx1, soff0, soff1,
                 gbuf0, gbuf1, obuf0, obuf1, sem0, sem1):
    wid = lax.axis_index("s") * NC + lax.axis_index("c")
    pbase = wid * (BPW // 2)
    pltpu.sync_copy(idx_hbm.at[wid], idx_v)

    hidx = (hidx0, hidx1)
    soff = (soff0, soff1)
    gbuf = (gbuf0, gbuf1)
    obuf = (obuf0, obuf1)
    sems = (sem0, sem1)

    def prep(t, b):
        for g in range(CH // 16):
            v = idx_v[t, pl.ds(g * 16, 16)]
            hidx[b][pl.ds(g * 16, 16)] = lax.shift_right_logical(v, 1)
            soff[b][pl.ds(g * 16, 16)] = lax.shift_left(v & 1, 6)

    def start(b):
        pltpu.async_copy(table_hbm.at[hidx[b]], gbuf[b], sems[b])

    def wait(b):
        pltpu.make_async_copy(table_hbm.at[hidx[b]], gbuf[b], sems[b]).wait()

    def select(b):
        def sel_body(g, carry):
            offs = soff[b][pl.ds(g * 16, 16)]
            for m in range(16):
                o = offs[m]
                kr = g * 16 + m
                orow = g * 8 + (m // 2)
                ocol = (m % 2) * D
                for j in range(D // 16):
                    obuf[b][orow, pl.ds(ocol + j * 16, 16)] = (
                        gbuf[b][kr, pl.ds(o + j * 16, 16)])
            return carry

        lax.fori_loop(0, CH // 16, sel_body, 0)

    prep(0, 0)
    start(0)
    prep(1, 1)
    start(1)

    def body(i, carry):
        for b in range(2):
            j = i * 2 + b
            wait(b)
            select(b)
            nj = j + 2

            @pl.when(nj < NCH)
            def _():
                prep(nj, b)
                start(b)

            pltpu.sync_copy(obuf[b], out_hbm.at[pl.ds(pbase + j * PCH, PCH)])
        return carry

    lax.fori_loop(0, NCH // 2, body, 0)


_SCRATCH = [
    pltpu.VMEM((NCH, CH), jnp.int32),
    pltpu.VMEM((CH,), jnp.int32),
    pltpu.VMEM((CH,), jnp.int32),
    pltpu.VMEM((CH,), jnp.int32),
    pltpu.VMEM((CH,), jnp.int32),
    pltpu.VMEM((CH, 2 * D), jnp.float32),
    pltpu.VMEM((CH, 2 * D), jnp.float32),
    pltpu.VMEM((PCH, 2 * D), jnp.float32),
    pltpu.VMEM((PCH, 2 * D), jnp.float32),
    pltpu.SemaphoreType.DMA,
    pltpu.SemaphoreType.DMA,
]

_gather_kernel = pl.kernel(
    _gather_body,
    mesh=_mesh,
    out_type=jax.ShapeDtypeStruct((B // 2, 2 * D), jnp.float32),
    scratch_types=_SCRATCH,
)


def kernel(x, weight):
    idx = x.reshape(NW, NCH, CH)
    table = weight.reshape(PV, 2 * D)
    out = _gather_kernel(idx, table)
    return out.reshape(x.shape[0], x.shape[1], D)
